# baseline (speedup 1.0000x reference)
import functools

import jax
import jax.numpy as jnp
from jax.experimental import pallas as pl
from jax.experimental.pallas import tpu as pltpu

GRID = 24
RES = 0.25
HALF = GRID * RES / 2.0
NCELL = GRID * GRID
NPAD = 640
CHUNK = 4096
F = 128
C1 = 64
C2 = 32
POOL = 4
PG = GRID // POOL
EMB = C2 * PG * PG


def _grid_kernel(x_ref, y_ref, v_ref, encw_ref, encb_ref,
                 w1_ref, b1_ref, w2_ref, b2_ref, out_ref, acc_ref,
                 *, num_chunks):
    i = pl.program_id(0)

    @pl.when(i == 0)
    def _init():
        acc_ref[...] = jnp.zeros_like(acc_ref)

    x = x_ref[0]
    y = y_ref[0]
    v = v_ref[0]
    cx = ((x + HALF) / RES).astype(jnp.int32)
    cy = ((y + HALF) / RES).astype(jnp.int32)
    mask = (cx >= 0) & (cx < GRID) & (cy >= 0) & (cy < GRID)
    idx = jnp.where(mask, cx * GRID + cy, NCELL)

    featT = jax.nn.relu(encw_ref[...] * v + encb_ref[...])
    feat_hi = featT.astype(jnp.bfloat16)
    feat_lo = (featT - feat_hi.astype(jnp.float32)).astype(jnp.bfloat16)

    cell_ids = jax.lax.broadcasted_iota(jnp.int32, (NPAD, CHUNK), 0)
    onehotT = (cell_ids == idx).astype(jnp.float32).astype(jnp.bfloat16)

    nt = (((1,), (1,)), ((), ()))
    acc_ref[...] += (
        jax.lax.dot_general(feat_hi, onehotT, nt,
                            preferred_element_type=jnp.float32) +
        jax.lax.dot_general(feat_lo, onehotT, nt,
                            preferred_element_type=jnp.float32))

    @pl.when(i == num_chunks - 1)
    def _tail():
        gt = acc_ref[:, :NCELL]

        r = jax.lax.broadcasted_iota(jnp.int32, (1, NCELL), 1)
        p = r // GRID
        q = r - p * GRID

        def conv(src, w_ref, b_ref, cin, cout):
            h = jnp.zeros((cout, NCELL), dtype=jnp.float32)
            for dd in range(9):
                di, dj = dd // 3 - 1, dd % 3 - 1
                off = di * GRID + dj
                valid = ((p + di >= 0) & (p + di < GRID) &
                         (q + dj >= 0) & (q + dj < GRID))
                shifted = jnp.roll(src, -off, axis=1) if off else src
                shifted = jnp.where(valid, shifted, 0.0)
                wdd = w_ref[dd * cout:(dd + 1) * cout, :]
                h = h + jnp.dot(wdd, shifted,
                                preferred_element_type=jnp.float32)
            return jax.nn.relu(h + b_ref[...])

        h1 = conv(gt, w1_ref, b1_ref, F, C1)
        h2 = conv(h1, w2_ref, b2_ref, C1, C2)

        rr = jax.lax.broadcasted_iota(jnp.int32, (NCELL, PG * PG), 0)
        ss = jax.lax.broadcasted_iota(jnp.int32, (NCELL, PG * PG), 1)
        pm = ((rr // (GRID * POOL) == ss // PG) &
              ((rr % GRID) // POOL == ss % PG))
        pmat = pm.astype(jnp.float32) * (1.0 / (POOL * POOL))
        out_ref[...] = jnp.dot(h2, pmat, preferred_element_type=jnp.float32)


def _heads_kernel(emb_ref,
                  cw1, cb1, cw2, cb2, cw3, cb3,
                  rw1, rb1, rw2, rb2, rw3, rb3,
                  sw1, sb1, sw2, sb2, sw3, sb3,
                  ow1, ob1, ow2, ob2, ow3, ob3,
                  clf_ref, reg_ref, sin_ref, cos_ref):
    emb = emb_ref[...]

    def head(w1, b1, w2, b2, w3, b3):
        h = jax.nn.relu(jnp.dot(emb, w1[...],
                                preferred_element_type=jnp.float32) + b1[...])
        h = jax.nn.relu(jnp.dot(h, w2[...],
                                preferred_element_type=jnp.float32) + b2[...])
        return jnp.dot(h, w3[...], preferred_element_type=jnp.float32) + b3[...]

    clf_ref[...] = head(cw1, cb1, cw2, cb2, cw3, cb3)
    reg_ref[...] = head(rw1, rb1, rw2, rb2, rw3, rb3)
    sin_ref[...] = jnp.tanh(head(sw1, sb1, sw2, sb2, sw3, sb3))
    cos_ref[...] = jnp.tanh(head(ow1, ob1, ow2, ob2, ow3, ob3))


def kernel(ventana, params):
    nwin, npts, _ = ventana.shape
    num_chunks = nwin * npts // CHUNK
    x = ventana[:, :, 0].reshape(num_chunks, 1, CHUNK)
    y = ventana[:, :, 1].reshape(num_chunks, 1, CHUNK)
    v = ventana[:, :, 3].reshape(num_chunks, 1, CHUNK)

    encw = params["enc"][0].reshape(1, F).T
    encb = params["enc"][1].reshape(1, F).T
    w1 = jnp.transpose(params["conv1"][0], (2, 3, 0, 1)).reshape(9 * C1, F)
    b1 = params["conv1"][1].reshape(C1, 1)
    w2 = jnp.transpose(params["conv2"][0], (2, 3, 0, 1)).reshape(9 * C2, C1)
    b2 = params["conv2"][1].reshape(C2, 1)

    chunk_spec = pl.BlockSpec((1, 1, CHUNK), lambda i: (i, 0, 0))
    full = lambda a: pl.BlockSpec(a.shape, lambda i: (0,) * a.ndim)

    pooled = pl.pallas_call(
        functools.partial(_grid_kernel, num_chunks=num_chunks),
        grid=(num_chunks,),
        in_specs=[chunk_spec, chunk_spec, chunk_spec,
                  full(encw), full(encb), full(w1), full(b1),
                  full(w2), full(b2)],
        out_specs=pl.BlockSpec((C2, PG * PG), lambda i: (0, 0)),
        out_shape=jax.ShapeDtypeStruct((C2, PG * PG), jnp.float32),
        scratch_shapes=[pltpu.VMEM((F, NPAD), jnp.float32)],
    )(x, y, v, encw, encb, w1, b1, w2, b2)

    emb = pooled.reshape(1, EMB)

    def hp(name):
        (w1_, b1_), (w2_, b2_), (w3_, b3_) = params[name]
        return (w1_, b1_.reshape(1, -1), w2_, b2_.reshape(1, -1),
                w3_, b3_.reshape(1, -1))

    head_args = hp("clf") + hp("reg") + hp("sin") + hp("cos")

    logits, reg_out, sin_out, cos_out = pl.pallas_call(
        _heads_kernel,
        out_shape=(jax.ShapeDtypeStruct((1, 8), jnp.float32),
                   jax.ShapeDtypeStruct((1, 6), jnp.float32),
                   jax.ShapeDtypeStruct((1, 1), jnp.float32),
                   jax.ShapeDtypeStruct((1, 1), jnp.float32)),
    )(emb, *head_args)

    cyc_out = jnp.concatenate([sin_out, cos_out], axis=1)
    return (logits, reg_out, cyc_out)


# speedup vs baseline: 1.3237x; 1.3237x over previous
import functools

import jax
import jax.numpy as jnp
from jax.experimental import pallas as pl
from jax.experimental.pallas import tpu as pltpu

GRID = 24
RES = 0.25
HALF = GRID * RES / 2.0
NCELL = GRID * GRID
NPAD = 640
CHUNK = 4096
F = 128
C1 = 64
C2 = 32
POOL = 4
PG = GRID // POOL
EMB = C2 * PG * PG


def _grid_kernel(x_ref, y_ref, v_ref, encw_ref, encb_ref,
                 w1_ref, b1_ref, w2_ref, b2_ref, out_ref, acc_ref,
                 *, num_chunks):
    i = pl.program_id(0)

    @pl.when(i == 0)
    def _init():
        acc_ref[...] = jnp.zeros_like(acc_ref)

    x = x_ref[0]
    y = y_ref[0]
    v = v_ref[0]
    cx = ((x + HALF) / RES).astype(jnp.int32)
    cy = ((y + HALF) / RES).astype(jnp.int32)
    mask = (cx >= 0) & (cx < GRID) & (cy >= 0) & (cy < GRID)
    idx = jnp.where(mask, cx * GRID + cy, NCELL)

    featT = jax.nn.relu(encw_ref[...] * v + encb_ref[...])

    cell_ids = jax.lax.broadcasted_iota(jnp.int32, (NPAD, CHUNK), 0)
    onehotT = (cell_ids == idx).astype(jnp.float32)

    acc_ref[...] += jax.lax.dot_general(
        featT, onehotT, (((1,), (1,)), ((), ())),
        preferred_element_type=jnp.float32)

    @pl.when(i == num_chunks - 1)
    def _tail():
        gt = acc_ref[:, :NCELL]

        r = jax.lax.broadcasted_iota(jnp.int32, (1, NCELL), 1)
        p = r // GRID
        q = r - p * GRID

        def conv(src, w_ref, b_ref, cin, cout):
            h = jnp.zeros((cout, NCELL), dtype=jnp.float32)
            for dd in range(9):
                di, dj = dd // 3 - 1, dd % 3 - 1
                off = di * GRID + dj
                valid = ((p + di >= 0) & (p + di < GRID) &
                         (q + dj >= 0) & (q + dj < GRID))
                shifted = jnp.roll(src, -off, axis=1) if off else src
                shifted = jnp.where(valid, shifted, 0.0)
                wdd = w_ref[dd * cout:(dd + 1) * cout, :]
                h = h + jnp.dot(wdd, shifted,
                                preferred_element_type=jnp.float32)
            return jax.nn.relu(h + b_ref[...])

        h1 = conv(gt, w1_ref, b1_ref, F, C1)
        h2 = conv(h1, w2_ref, b2_ref, C1, C2)

        rr = jax.lax.broadcasted_iota(jnp.int32, (NCELL, PG * PG), 0)
        ss = jax.lax.broadcasted_iota(jnp.int32, (NCELL, PG * PG), 1)
        pm = ((rr // (GRID * POOL) == ss // PG) &
              ((rr % GRID) // POOL == ss % PG))
        pmat = pm.astype(jnp.float32) * (1.0 / (POOL * POOL))
        out_ref[...] = jnp.dot(h2, pmat, preferred_element_type=jnp.float32)


def _heads_kernel(emb_ref,
                  cw1, cb1, cw2, cb2, cw3, cb3,
                  rw1, rb1, rw2, rb2, rw3, rb3,
                  sw1, sb1, sw2, sb2, sw3, sb3,
                  ow1, ob1, ow2, ob2, ow3, ob3,
                  clf_ref, reg_ref, sin_ref, cos_ref):
    emb = emb_ref[...]

    def head(w1, b1, w2, b2, w3, b3):
        h = jax.nn.relu(jnp.dot(emb, w1[...],
                                preferred_element_type=jnp.float32) + b1[...])
        h = jax.nn.relu(jnp.dot(h, w2[...],
                                preferred_element_type=jnp.float32) + b2[...])
        return jnp.dot(h, w3[...], preferred_element_type=jnp.float32) + b3[...]

    clf_ref[...] = head(cw1, cb1, cw2, cb2, cw3, cb3)
    reg_ref[...] = head(rw1, rb1, rw2, rb2, rw3, rb3)
    sin_ref[...] = jnp.tanh(head(sw1, sb1, sw2, sb2, sw3, sb3))
    cos_ref[...] = jnp.tanh(head(ow1, ob1, ow2, ob2, ow3, ob3))


def kernel(ventana, params):
    nwin, npts, _ = ventana.shape
    num_chunks = nwin * npts // CHUNK
    x = ventana[:, :, 0].reshape(num_chunks, 1, CHUNK)
    y = ventana[:, :, 1].reshape(num_chunks, 1, CHUNK)
    v = ventana[:, :, 3].reshape(num_chunks, 1, CHUNK)

    encw = params["enc"][0].reshape(1, F).T
    encb = params["enc"][1].reshape(1, F).T
    w1 = jnp.transpose(params["conv1"][0], (2, 3, 0, 1)).reshape(9 * C1, F)
    b1 = params["conv1"][1].reshape(C1, 1)
    w2 = jnp.transpose(params["conv2"][0], (2, 3, 0, 1)).reshape(9 * C2, C1)
    b2 = params["conv2"][1].reshape(C2, 1)

    chunk_spec = pl.BlockSpec((1, 1, CHUNK), lambda i: (i, 0, 0))
    full = lambda a: pl.BlockSpec(a.shape, lambda i: (0,) * a.ndim)

    pooled = pl.pallas_call(
        functools.partial(_grid_kernel, num_chunks=num_chunks),
        grid=(num_chunks,),
        in_specs=[chunk_spec, chunk_spec, chunk_spec,
                  full(encw), full(encb), full(w1), full(b1),
                  full(w2), full(b2)],
        out_specs=pl.BlockSpec((C2, PG * PG), lambda i: (0, 0)),
        out_shape=jax.ShapeDtypeStruct((C2, PG * PG), jnp.float32),
        scratch_shapes=[pltpu.VMEM((F, NPAD), jnp.float32)],
    )(x, y, v, encw, encb, w1, b1, w2, b2)

    emb = pooled.reshape(1, EMB)

    def hp(name):
        (w1_, b1_), (w2_, b2_), (w3_, b3_) = params[name]
        return (w1_, b1_.reshape(1, -1), w2_, b2_.reshape(1, -1),
                w3_, b3_.reshape(1, -1))

    head_args = hp("clf") + hp("reg") + hp("sin") + hp("cos")

    logits, reg_out, sin_out, cos_out = pl.pallas_call(
        _heads_kernel,
        out_shape=(jax.ShapeDtypeStruct((1, 8), jnp.float32),
                   jax.ShapeDtypeStruct((1, 6), jnp.float32),
                   jax.ShapeDtypeStruct((1, 1), jnp.float32),
                   jax.ShapeDtypeStruct((1, 1), jnp.float32)),
    )(emb, *head_args)

    cyc_out = jnp.concatenate([sin_out, cos_out], axis=1)
    return (logits, reg_out, cyc_out)


# speedup vs baseline: 1.3759x; 1.0395x over previous
import functools

import jax
import jax.numpy as jnp
from jax.experimental import pallas as pl
from jax.experimental.pallas import tpu as pltpu

GRID = 24
RES = 0.25
HALF = GRID * RES / 2.0
NCELL = GRID * GRID
NPAD = 640
CHUNK = 4096
F = 128
C1 = 64
C2 = 32
POOL = 4
PG = GRID // POOL
EMB = C2 * PG * PG


def _fused_kernel(x_ref, y_ref, v_ref, encw_ref, encb_ref,
                  w1_ref, b1_ref, w2_ref, b2_ref,
                  cw1, cb1, cw2, cb2, cw3, cb3,
                  rw1, rb1, rw2, rb2, rw3, rb3,
                  sw1, sb1, sw2, sb2, sw3, sb3,
                  ow1, ob1, ow2, ob2, ow3, ob3,
                  clf_ref, reg_ref, cyc_ref, acc_ref,
                  *, num_chunks):
    i = pl.program_id(0)

    @pl.when(i == 0)
    def _init():
        acc_ref[...] = jnp.zeros_like(acc_ref)

    x = x_ref[0]
    y = y_ref[0]
    v = v_ref[0]
    cx = ((x + HALF) / RES).astype(jnp.int32)
    cy = ((y + HALF) / RES).astype(jnp.int32)
    mask = (cx >= 0) & (cx < GRID) & (cy >= 0) & (cy < GRID)
    idx = jnp.where(mask, cx * GRID + cy, NCELL)

    featT = jax.nn.relu(encw_ref[...] * v + encb_ref[...])

    cell_ids = jax.lax.broadcasted_iota(jnp.int32, (NPAD, CHUNK), 0)
    onehotT = (cell_ids == idx).astype(jnp.float32)

    acc_ref[...] += jax.lax.dot_general(
        featT, onehotT, (((1,), (1,)), ((), ())),
        preferred_element_type=jnp.float32)

    @pl.when(i == num_chunks - 1)
    def _tail():
        gt = acc_ref[:, :NCELL]

        r = jax.lax.broadcasted_iota(jnp.int32, (1, NCELL), 1)
        p = r // GRID
        q = r - p * GRID

        def conv(src, w_ref, b_ref, cout):
            h = jnp.zeros((cout, NCELL), dtype=jnp.float32)
            for dd in range(9):
                di, dj = dd // 3 - 1, dd % 3 - 1
                off = di * GRID + dj
                valid = ((p + di >= 0) & (p + di < GRID) &
                         (q + dj >= 0) & (q + dj < GRID))
                shifted = jnp.roll(src, -off, axis=1) if off else src
                shifted = jnp.where(valid, shifted, 0.0)
                wdd = w_ref[dd * cout:(dd + 1) * cout, :]
                h = h + jnp.dot(wdd, shifted,
                                preferred_element_type=jnp.float32)
            return jax.nn.relu(h + b_ref[...])

        h1 = conv(gt, w1_ref, b1_ref, C1)
        h2 = conv(h1, w2_ref, b2_ref, C2)

        rr = jax.lax.broadcasted_iota(jnp.int32, (NCELL, PG * PG), 0)
        ss = jax.lax.broadcasted_iota(jnp.int32, (NCELL, PG * PG), 1)
        pm = ((rr // (GRID * POOL) == ss // PG) &
              ((rr % GRID) // POOL == ss % PG))
        pmat = pm.astype(jnp.float32) * (1.0 / (POOL * POOL))
        pooled = jnp.dot(h2, pmat, preferred_element_type=jnp.float32)

        emb = jnp.concatenate([pooled[c:c + 1, :] for c in range(C2)], axis=1)

        def head(w1, b1, w2, b2, w3, b3):
            h = jax.nn.relu(jnp.dot(emb, w1[...],
                                    preferred_element_type=jnp.float32)
                            + b1[...])
            h = jax.nn.relu(jnp.dot(h, w2[...],
                                    preferred_element_type=jnp.float32)
                            + b2[...])
            return jnp.dot(h, w3[...],
                           preferred_element_type=jnp.float32) + b3[...]

        clf_ref[...] = head(cw1, cb1, cw2, cb2, cw3, cb3)
        reg_ref[...] = head(rw1, rb1, rw2, rb2, rw3, rb3)
        sin_o = jnp.tanh(head(sw1, sb1, sw2, sb2, sw3, sb3))
        cos_o = jnp.tanh(head(ow1, ob1, ow2, ob2, ow3, ob3))
        cyc_ref[...] = jnp.concatenate([sin_o, cos_o], axis=1)


def kernel(ventana, params):
    nwin, npts, _ = ventana.shape
    num_chunks = nwin * npts // CHUNK
    x = ventana[:, :, 0].reshape(num_chunks, 1, CHUNK)
    y = ventana[:, :, 1].reshape(num_chunks, 1, CHUNK)
    v = ventana[:, :, 3].reshape(num_chunks, 1, CHUNK)

    encw = params["enc"][0].reshape(1, F).T
    encb = params["enc"][1].reshape(1, F).T
    w1 = jnp.transpose(params["conv1"][0], (2, 3, 0, 1)).reshape(9 * C1, F)
    b1 = params["conv1"][1].reshape(C1, 1)
    w2 = jnp.transpose(params["conv2"][0], (2, 3, 0, 1)).reshape(9 * C2, C1)
    b2 = params["conv2"][1].reshape(C2, 1)

    def hp(name):
        (w1_, b1_), (w2_, b2_), (w3_, b3_) = params[name]
        return (w1_, b1_.reshape(1, -1), w2_, b2_.reshape(1, -1),
                w3_, b3_.reshape(1, -1))

    head_args = hp("clf") + hp("reg") + hp("sin") + hp("cos")

    chunk_spec = pl.BlockSpec((1, 1, CHUNK), lambda i: (i, 0, 0))
    full = lambda a: pl.BlockSpec(a.shape, lambda i: (0,) * a.ndim)
    weights = (encw, encb, w1, b1, w2, b2) + head_args

    logits, reg_out, cyc_out = pl.pallas_call(
        functools.partial(_fused_kernel, num_chunks=num_chunks),
        grid=(num_chunks,),
        in_specs=[chunk_spec, chunk_spec, chunk_spec] +
                 [full(a) for a in weights],
        out_specs=(pl.BlockSpec((1, 8), lambda i: (0, 0)),
                   pl.BlockSpec((1, 6), lambda i: (0, 0)),
                   pl.BlockSpec((1, 2), lambda i: (0, 0))),
        out_shape=(jax.ShapeDtypeStruct((1, 8), jnp.float32),
                   jax.ShapeDtypeStruct((1, 6), jnp.float32),
                   jax.ShapeDtypeStruct((1, 2), jnp.float32)),
        scratch_shapes=[pltpu.VMEM((F, NPAD), jnp.float32)],
    )(x, y, v, *weights)

    return (logits, reg_out, cyc_out)
